# baseline (speedup 1.0000x reference)
import functools

import jax
import jax.numpy as jnp
import numpy as np
from jax import lax
from jax.experimental import pallas as pl
from jax.experimental.pallas import tpu as pltpu
from jax.experimental.pallas import tpu_sc as plsc

_NUM_PATHS = 3200
_BATCH = 16
_TOTAL_ARCS = 819200
_L_HYP = 128
_L_REF = 128
_LIMBS = 8
_NW = 32
_ROWS = _TOTAL_ARCS // _NW // 128

_W_np = np.zeros((_LIMBS, _L_REF), np.float32)
for _l in range(_LIMBS):
    for _e in range(16):
        _W_np[_l, 16 * _l + _e] = float(2 ** _e)


def _sc_segsum(arc_scores, arc_to_path, zeros_init):
    scores2d = arc_scores.reshape(_NW * _ROWS, 128)
    idx2d = arc_to_path.reshape(_NW * _ROWS, 128)
    mesh = plsc.VectorSubcoreMesh(core_axis_name="c", subcore_axis_name="s")

    @functools.partial(
        pl.kernel,
        mesh=mesh,
        out_type=jax.ShapeDtypeStruct((2, _NUM_PATHS), jnp.float32),
        scratch_types=[
            pltpu.VMEM((_ROWS, 128), jnp.float32),
            pltpu.VMEM((_ROWS, 128), jnp.int32),
            pltpu.VMEM_SHARED((_NUM_PATHS,), jnp.float32),
            pltpu.SemaphoreType.DMA,
        ],
    )
    def seg_kernel(scores_hbm, idx_hbm, zeros_hbm, out_hbm, vals_v, idx_v,
                   acc_sh, sem):
        c = lax.axis_index("c")
        s = lax.axis_index("s")
        w = c * 16 + s

        @pl.when(s == 0)
        def _zero():
            pltpu.sync_copy(zeros_hbm, acc_sh)

        base = w * _ROWS
        pltpu.sync_copy(scores_hbm.at[pl.ds(base, _ROWS)], vals_v)
        pltpu.sync_copy(idx_hbm.at[pl.ds(base, _ROWS)], idx_v)
        plsc.subcore_barrier()

        def start_body(j, carry):
            pltpu.async_copy(vals_v.at[j], acc_sh.at[idx_v.at[j]], sem,
                             add=True)
            return carry

        lax.fori_loop(0, _ROWS, start_body, 0)

        def wait_body(j, carry):
            pltpu.make_async_copy(vals_v.at[j], acc_sh.at[idx_v.at[j]],
                                  sem).wait()
            return carry

        lax.fori_loop(0, _ROWS, wait_body, 0)
        plsc.subcore_barrier()

        @pl.when(s == 0)
        def _out():
            pltpu.sync_copy(acc_sh, out_hbm.at[c])

    return seg_kernel(scores2d, idx2d, zeros_init)


_M16 = 0xFFFF


def _up(x, k):
    return jnp.concatenate(
        [jnp.zeros((k, x.shape[1]), jnp.int32), x[:-k, :]], axis=0)


def _add128(a, b):
    s = a + b
    g = s >> 16
    s = s & _M16
    p = (s + 1) >> 16
    g = g | (p & _up(g, 1))
    p = p & _up(p, 1)
    g = g | (p & _up(g, 2))
    p = p & _up(p, 2)
    g = g | (p & _up(g, 4))
    return (s + _up(g, 1)) & _M16


def _sl1(x):
    return ((x << 1) & _M16) | _up(x >> 15, 1)


def _not16(x):
    return x ^ _M16


_CPB = 8


def _dp_body(hyp_ref, utt_ref, hlen_ref, refT_ref, rlen_ref,
             w_ref, out_ref, refpp, vp_r, vn_r, score_r, capt_r, maskrl_r,
             c1_r):
    i = pl.program_id(0)
    P = _NUM_PATHS

    @pl.when(i == 0)
    def _init():
        utt = utt_ref[0:1, :]
        rp = jnp.zeros((_L_REF, P), jnp.int32)
        rl = jnp.zeros((1, P), jnp.int32)
        for u in range(_BATCH):
            m = utt == u
            rp = jnp.where(m, refT_ref[:, u:u + 1], rp)
            rl = jnp.where(m, rlen_ref[0, u], rl)
        refpp[:, :] = rp
        liota = lax.broadcasted_iota(jnp.int32, (_LIMBS, P), 0)
        rlm1 = rl - 1
        limb = rlm1 >> 4
        bit = rlm1 & 15
        maskval = jnp.left_shift(jnp.ones_like(bit), bit)
        maskrl_r[:, :] = jnp.where(liota == limb, maskval, 0)
        c1_r[:, :] = jnp.where(liota == 0, 1, 0)
        score_r[:, :] = jnp.where(liota == 0, rl, 0)
        capt_r[:, :] = jnp.zeros((_LIMBS, P), jnp.int32)
        vp_r[:, :] = jnp.full((_LIMBS, P), 0xFFFF, jnp.int32)
        vn_r[:, :] = jnp.zeros((_LIMBS, P), jnp.int32)

    vp = vp_r[:, :]
    vn = vn_r[:, :]
    score = score_r[:, :]
    capt = capt_r[:, :]
    mrl = maskrl_r[:, :]
    c1 = c1_r[:, :]
    rp = refpp[:, :]
    w = w_ref[:, :]
    hlen = hlen_ref[0:1, :]

    for k in range(_CPB):
        hyp_i = hyp_ref[k, 0:1, :]
        e_f = jnp.where(rp == hyp_i, 1.0, 0.0).astype(jnp.float32)
        eq = lax.dot_general(w, e_f, (((1,), (0,)), ((), ())),
                             preferred_element_type=jnp.float32
                             ).astype(jnp.int32)

        xv = eq | vn
        xh = (_add128(eq & vp, vp) ^ vp) | eq
        hp = vn | _not16(xh | vp)
        hn = vp & xh

        dplus = jnp.where((hp & mrl) != 0, 1, 0)
        dminus = jnp.where((hn & mrl) != 0, 1, 0)
        score = score + dplus - dminus
        hit = hlen == (i * _CPB + k + 1)
        capt = jnp.where(hit, score, capt)

        hps = _sl1(hp) | c1
        hns = _sl1(hn)
        vp = hns | _not16(xv | hps)
        vn = hps & xv

    vp_r[:, :] = vp
    vn_r[:, :] = vn
    score_r[:, :] = score
    capt_r[:, :] = capt

    @pl.when(i == _L_HYP // _CPB - 1)
    def _fin():
        out_ref[:, :] = jnp.sum(capt, axis=0, keepdims=True
                                ).astype(jnp.float32)


def _final_body(partial_ref, wers_ref, utt_ref, out_ref):
    P = _NUM_PATHS
    wers = wers_ref[:, :]
    plp = partial_ref[0:1, :] + partial_ref[1:2, :]
    pprob = jnp.exp(plp)
    utt = utt_ref[:, :]
    den = jnp.zeros((1, P), jnp.float32)
    for u in range(_BATCH):
        m = utt == u
        du = jnp.sum(jnp.where(m, pprob, 0.0))
        den = jnp.where(m, du, den)
    dlp = jnp.log(den)
    out_ref[:, :] = jnp.sum(jnp.exp(plp - dlp) * wers, axis=1,
                            keepdims=True)


def _tc_final(partials, wers, utt2):
    P = _NUM_PATHS
    return pl.pallas_call(
        _final_body,
        out_shape=jax.ShapeDtypeStruct((1, 1), jnp.float32),
    )(partials, wers, utt2)


def _tc_mwer(hyp_T3, utt2, hlen2, refT, rlen2, wmat):
    P = _NUM_PATHS
    return pl.pallas_call(
        _dp_body,
        grid=(_L_HYP // _CPB,),
        in_specs=[
            pl.BlockSpec((_CPB, 1, P), lambda i: (i, 0, 0)),
            pl.BlockSpec((1, P), lambda i: (0, 0)),
            pl.BlockSpec((1, P), lambda i: (0, 0)),
            pl.BlockSpec((_L_REF, _BATCH), lambda i: (0, 0)),
            pl.BlockSpec((1, _BATCH), lambda i: (0, 0)),
            pl.BlockSpec((_LIMBS, _L_REF), lambda i: (0, 0)),
        ],
        out_specs=pl.BlockSpec((1, P), lambda i: (0, 0)),
        out_shape=jax.ShapeDtypeStruct((1, P), jnp.float32),
        scratch_shapes=[
            pltpu.VMEM((_L_REF, P), jnp.int32),
            pltpu.VMEM((_LIMBS, P), jnp.int32),
            pltpu.VMEM((_LIMBS, P), jnp.int32),
            pltpu.VMEM((_LIMBS, P), jnp.int32),
            pltpu.VMEM((_LIMBS, P), jnp.int32),
            pltpu.VMEM((_LIMBS, P), jnp.int32),
            pltpu.VMEM((_LIMBS, P), jnp.int32),
        ],
        compiler_params=pltpu.CompilerParams(
            dimension_semantics=("arbitrary",)),
    )(hyp_T3, utt2, hlen2, refT, rlen2, wmat)


def kernel(arc_scores, arc_to_path, path_to_utt, hyp_tokens, hyp_lens,
           ref_tokens, ref_lens, num_paths, nbest_scale):
    del num_paths, nbest_scale
    partials = _sc_segsum(arc_scores, arc_to_path.astype(jnp.int32),
                          jnp.zeros((_NUM_PATHS,), jnp.float32))
    hyp_T3 = hyp_tokens.T.reshape(_L_HYP, 1, _NUM_PATHS)
    utt2 = path_to_utt.reshape(1, _NUM_PATHS)
    wers = _tc_mwer(
        hyp_T3,
        utt2,
        hyp_lens.reshape(1, _NUM_PATHS),
        ref_tokens.T,
        ref_lens.reshape(1, _BATCH),
        jnp.asarray(_W_np),
    )
    out = _tc_final(partials, wers, utt2)
    return out[0, 0]


# speedup vs baseline: 1.0133x; 1.0133x over previous
import functools

import jax
import jax.numpy as jnp
import numpy as np
from jax import lax
from jax.experimental import pallas as pl
from jax.experimental.pallas import tpu as pltpu
from jax.experimental.pallas import tpu_sc as plsc

_NUM_PATHS = 3200
_BATCH = 16
_TOTAL_ARCS = 819200
_L_HYP = 128
_L_REF = 128
_LIMBS = 8
_NW = 32
_ROWS = _TOTAL_ARCS // _NW // 128

_W_np = np.zeros((_LIMBS, _L_REF), np.float32)
for _l in range(_LIMBS):
    for _e in range(16):
        _W_np[_l, 16 * _l + _e] = float(2 ** _e)


def _sc_segsum(arc_scores, arc_to_path, zeros_init):
    scores2d = arc_scores.reshape(_NW * _ROWS, 128)
    idx2d = arc_to_path.reshape(_NW * _ROWS, 128)
    mesh = plsc.VectorSubcoreMesh(core_axis_name="c", subcore_axis_name="s")

    @functools.partial(
        pl.kernel,
        mesh=mesh,
        out_type=jax.ShapeDtypeStruct((2, _NUM_PATHS), jnp.float32),
        scratch_types=[
            pltpu.VMEM((_ROWS, 128), jnp.float32),
            pltpu.VMEM((_ROWS, 128), jnp.int32),
            pltpu.VMEM_SHARED((_NUM_PATHS,), jnp.float32),
            pltpu.SemaphoreType.DMA,
        ],
    )
    def seg_kernel(scores_hbm, idx_hbm, zeros_hbm, out_hbm, vals_v, idx_v,
                   acc_sh, sem):
        c = lax.axis_index("c")
        s = lax.axis_index("s")
        w = c * 16 + s

        @pl.when(s == 0)
        def _zero():
            pltpu.sync_copy(zeros_hbm, acc_sh)

        base = w * _ROWS
        pltpu.sync_copy(scores_hbm.at[pl.ds(base, _ROWS)], vals_v)
        pltpu.sync_copy(idx_hbm.at[pl.ds(base, _ROWS)], idx_v)
        plsc.subcore_barrier()

        def start_body(j, carry):
            pltpu.async_copy(vals_v.at[j], acc_sh.at[idx_v.at[j]], sem,
                             add=True)
            return carry

        lax.fori_loop(0, _ROWS, start_body, 0)

        def wait_body(j, carry):
            pltpu.make_async_copy(vals_v.at[j], acc_sh.at[idx_v.at[j]],
                                  sem).wait()
            return carry

        lax.fori_loop(0, _ROWS, wait_body, 0)
        plsc.subcore_barrier()

        @pl.when(s == 0)
        def _out():
            pltpu.sync_copy(acc_sh, out_hbm.at[c])

    return seg_kernel(scores2d, idx2d, zeros_init)


_M16 = 0xFFFF


def _up(x, k):
    return jnp.concatenate(
        [jnp.zeros((k, x.shape[1]), jnp.int32), x[:-k, :]], axis=0)


def _add128(a, b):
    s = a + b
    g = s >> 16
    s = s & _M16
    p = (s + 1) >> 16
    g = g | (p & _up(g, 1))
    p = p & _up(p, 1)
    g = g | (p & _up(g, 2))
    p = p & _up(p, 2)
    g = g | (p & _up(g, 4))
    return (s + _up(g, 1)) & _M16


def _sl1(x):
    return ((x << 1) & _M16) | _up(x >> 15, 1)


def _not16(x):
    return x ^ _M16


_CPB = 16


def _dp_body(hyp_ref, utt_ref, hlen_ref, refT_ref, rlen_ref,
             w_ref, out_ref, refpp, vp_r, vn_r, score_r, capt_r, maskrl_r,
             c1_r):
    i = pl.program_id(0)
    P = _NUM_PATHS

    @pl.when(i == 0)
    def _init():
        utt = utt_ref[0:1, :]
        rp = jnp.zeros((_L_REF, P), jnp.int32)
        rl = jnp.zeros((1, P), jnp.int32)
        for u in range(_BATCH):
            m = utt == u
            rp = jnp.where(m, refT_ref[:, u:u + 1], rp)
            rl = jnp.where(m, rlen_ref[0, u], rl)
        refpp[:, :] = rp
        liota = lax.broadcasted_iota(jnp.int32, (_LIMBS, P), 0)
        rlm1 = rl - 1
        limb = rlm1 >> 4
        bit = rlm1 & 15
        maskval = jnp.left_shift(jnp.ones_like(bit), bit)
        maskrl_r[:, :] = jnp.where(liota == limb, maskval, 0)
        c1_r[:, :] = jnp.where(liota == 0, 1, 0)
        score_r[:, :] = jnp.where(liota == 0, rl, 0)
        capt_r[:, :] = jnp.zeros((_LIMBS, P), jnp.int32)
        vp_r[:, :] = jnp.full((_LIMBS, P), 0xFFFF, jnp.int32)
        vn_r[:, :] = jnp.zeros((_LIMBS, P), jnp.int32)

    vp = vp_r[:, :]
    vn = vn_r[:, :]
    score = score_r[:, :]
    capt = capt_r[:, :]
    mrl = maskrl_r[:, :]
    c1 = c1_r[:, :]
    rp = refpp[:, :]
    w = w_ref[:, :]
    hlen = hlen_ref[0:1, :]

    for k in range(_CPB):
        hyp_i = hyp_ref[k, 0:1, :]
        e_f = jnp.where(rp == hyp_i, 1.0, 0.0).astype(jnp.float32)
        eq = lax.dot_general(w, e_f, (((1,), (0,)), ((), ())),
                             preferred_element_type=jnp.float32
                             ).astype(jnp.int32)

        xv = eq | vn
        xh = (_add128(eq & vp, vp) ^ vp) | eq
        hp = vn | _not16(xh | vp)
        hn = vp & xh

        dplus = jnp.where((hp & mrl) != 0, 1, 0)
        dminus = jnp.where((hn & mrl) != 0, 1, 0)
        score = score + dplus - dminus
        hit = hlen == (i * _CPB + k + 1)
        capt = jnp.where(hit, score, capt)

        hps = _sl1(hp) | c1
        hns = _sl1(hn)
        vp = hns | _not16(xv | hps)
        vn = hps & xv

    vp_r[:, :] = vp
    vn_r[:, :] = vn
    score_r[:, :] = score
    capt_r[:, :] = capt

    @pl.when(i == _L_HYP // _CPB - 1)
    def _fin():
        out_ref[:, :] = jnp.sum(capt, axis=0, keepdims=True
                                ).astype(jnp.float32)


def _final_body(partial_ref, wers_ref, utt_ref, out_ref):
    P = _NUM_PATHS
    wers = wers_ref[:, :]
    plp = partial_ref[0:1, :] + partial_ref[1:2, :]
    pprob = jnp.exp(plp)
    utt = utt_ref[:, :]
    den = jnp.zeros((1, P), jnp.float32)
    for u in range(_BATCH):
        m = utt == u
        du = jnp.sum(jnp.where(m, pprob, 0.0))
        den = jnp.where(m, du, den)
    dlp = jnp.log(den)
    out_ref[:, :] = jnp.sum(jnp.exp(plp - dlp) * wers, axis=1,
                            keepdims=True)


def _tc_final(partials, wers, utt2):
    P = _NUM_PATHS
    return pl.pallas_call(
        _final_body,
        out_shape=jax.ShapeDtypeStruct((1, 1), jnp.float32),
    )(partials, wers, utt2)


def _tc_mwer(hyp_T3, utt2, hlen2, refT, rlen2, wmat):
    P = _NUM_PATHS
    return pl.pallas_call(
        _dp_body,
        grid=(_L_HYP // _CPB,),
        in_specs=[
            pl.BlockSpec((_CPB, 1, P), lambda i: (i, 0, 0)),
            pl.BlockSpec((1, P), lambda i: (0, 0)),
            pl.BlockSpec((1, P), lambda i: (0, 0)),
            pl.BlockSpec((_L_REF, _BATCH), lambda i: (0, 0)),
            pl.BlockSpec((1, _BATCH), lambda i: (0, 0)),
            pl.BlockSpec((_LIMBS, _L_REF), lambda i: (0, 0)),
        ],
        out_specs=pl.BlockSpec((1, P), lambda i: (0, 0)),
        out_shape=jax.ShapeDtypeStruct((1, P), jnp.float32),
        scratch_shapes=[
            pltpu.VMEM((_L_REF, P), jnp.int32),
            pltpu.VMEM((_LIMBS, P), jnp.int32),
            pltpu.VMEM((_LIMBS, P), jnp.int32),
            pltpu.VMEM((_LIMBS, P), jnp.int32),
            pltpu.VMEM((_LIMBS, P), jnp.int32),
            pltpu.VMEM((_LIMBS, P), jnp.int32),
            pltpu.VMEM((_LIMBS, P), jnp.int32),
        ],
        compiler_params=pltpu.CompilerParams(
            dimension_semantics=("arbitrary",)),
    )(hyp_T3, utt2, hlen2, refT, rlen2, wmat)


def kernel(arc_scores, arc_to_path, path_to_utt, hyp_tokens, hyp_lens,
           ref_tokens, ref_lens, num_paths, nbest_scale):
    del num_paths, nbest_scale
    partials = _sc_segsum(arc_scores, arc_to_path.astype(jnp.int32),
                          jnp.zeros((_NUM_PATHS,), jnp.float32))
    hyp_T3 = hyp_tokens.T.reshape(_L_HYP, 1, _NUM_PATHS)
    utt2 = path_to_utt.reshape(1, _NUM_PATHS)
    wers = _tc_mwer(
        hyp_T3,
        utt2,
        hyp_lens.reshape(1, _NUM_PATHS),
        ref_tokens.T,
        ref_lens.reshape(1, _BATCH),
        jnp.asarray(_W_np),
    )
    out = _tc_final(partials, wers, utt2)
    return out[0, 0]
